# baseline (speedup 1.0000x reference)
import jax
import jax.numpy as jnp
from jax.experimental import pallas as pl
from jax.experimental.pallas import tpu as pltpu

B = 16384
D_IN = 128
D_OUT = 128
CHUNK = 2048
NBUF = 3
NCHUNK = B // CHUNK
_Q = CHUNK // 128


def _in_copy(data_hbm, dbuf, isem, i):
    buf = i % NBUF
    return pltpu.make_async_copy(
        data_hbm.at[pl.ds(i * CHUNK, CHUNK), :], dbuf.at[buf], isem.at[buf]
    )


def _out_copy(o_hbm, obuf, osem, i):
    buf = i % NBUF
    return pltpu.make_async_copy(
        obuf.at[buf], o_hbm.at[pl.ds(i * CHUNK, CHUNK), :], osem.at[buf]
    )


def _body(ind_hbm, data_hbm, w_hbm, b_hbm, o_hbm,
          ind_v, w_v, b_v, dbuf, obuf, csem, isem, osem):
    cp_ind = pltpu.make_async_copy(ind_hbm, ind_v, csem.at[0])
    cp_w = pltpu.make_async_copy(w_hbm, w_v, csem.at[1])
    cp_b = pltpu.make_async_copy(b_hbm, b_v, csem.at[2])
    cp_ind.start()
    cp_w.start()
    cp_b.start()
    for k in range(min(NBUF, NCHUNK)):
        _in_copy(data_hbm, dbuf, isem, k).start()
    cp_ind.wait()
    cp_w.wait()
    cp_b.wait()
    for i in range(NCHUNK):
        buf = i % NBUF
        _in_copy(data_hbm, dbuf, isem, i).wait()
        if i >= NBUF:
            _out_copy(o_hbm, obuf, osem, i - NBUF).wait()
        x = dbuf[buf]
        y = jnp.dot(x, w_v[...], preferred_element_type=jnp.float32)
        y = y + b_v[...]
        ind = ind_v[pl.ds(i * _Q, _Q), :]
        y3 = y.reshape(_Q, 128, D_OUT)
        masked = jnp.where(ind[:, :, None] != 0, y3, 0.0)
        obuf[buf] = masked.reshape(CHUNK, D_OUT)
        _out_copy(o_hbm, obuf, osem, i).start()
        if i + NBUF < NCHUNK:
            _in_copy(data_hbm, dbuf, isem, i + NBUF).start()
    for i in range(max(0, NCHUNK - NBUF), NCHUNK):
        _out_copy(o_hbm, obuf, osem, i).wait()


def kernel(indicators, data, W, b):
    ind2d = indicators.reshape(128, 128)
    b2d = b.reshape(1, D_OUT)
    hbm = pl.BlockSpec(memory_space=pltpu.MemorySpace.HBM)
    return pl.pallas_call(
        _body,
        in_specs=[hbm, hbm, hbm, hbm],
        out_specs=hbm,
        out_shape=jax.ShapeDtypeStruct((B, D_OUT), jnp.float32),
        scratch_shapes=[
            pltpu.VMEM((128, 128), jnp.int32),
            pltpu.VMEM((D_IN, D_OUT), jnp.float32),
            pltpu.VMEM((1, D_OUT), jnp.float32),
            pltpu.VMEM((NBUF, CHUNK, D_IN), jnp.float32),
            pltpu.VMEM((NBUF, CHUNK, D_OUT), jnp.float32),
            pltpu.SemaphoreType.DMA((3,)),
            pltpu.SemaphoreType.DMA((NBUF,)),
            pltpu.SemaphoreType.DMA((NBUF,)),
        ],
    )(ind2d, data, W, b2d)


# speedup vs baseline: 1.0178x; 1.0178x over previous
import jax
import jax.numpy as jnp
from jax.experimental import pallas as pl
from jax.experimental.pallas import tpu as pltpu

B = 16384
D_IN = 128
D_OUT = 128
CHUNK = 4096
NBUF = 3
NCHUNK = B // CHUNK
_Q = CHUNK // 128


def _in_copy(data_hbm, dbuf, isem, i):
    buf = i % NBUF
    return pltpu.make_async_copy(
        data_hbm.at[pl.ds(i * CHUNK, CHUNK), :], dbuf.at[buf], isem.at[buf]
    )


def _out_copy(o_hbm, obuf, osem, i):
    buf = i % NBUF
    return pltpu.make_async_copy(
        obuf.at[buf], o_hbm.at[pl.ds(i * CHUNK, CHUNK), :], osem.at[buf]
    )


def _body(ind_hbm, data_hbm, w_hbm, b_hbm, o_hbm,
          ind_v, w_v, b_v, dbuf, obuf, csem, isem, osem):
    cp_ind = pltpu.make_async_copy(ind_hbm, ind_v, csem.at[0])
    cp_w = pltpu.make_async_copy(w_hbm, w_v, csem.at[1])
    cp_b = pltpu.make_async_copy(b_hbm, b_v, csem.at[2])
    cp_ind.start()
    cp_w.start()
    cp_b.start()
    for k in range(min(NBUF, NCHUNK)):
        _in_copy(data_hbm, dbuf, isem, k).start()
    cp_ind.wait()
    cp_w.wait()
    cp_b.wait()
    for i in range(NCHUNK):
        buf = i % NBUF
        _in_copy(data_hbm, dbuf, isem, i).wait()
        if i >= NBUF:
            _out_copy(o_hbm, obuf, osem, i - NBUF).wait()
        x = dbuf[buf]
        y = jnp.dot(x, w_v[...], preferred_element_type=jnp.float32)
        y = y + b_v[...]
        ind = ind_v[pl.ds(i * _Q, _Q), :]
        y3 = y.reshape(_Q, 128, D_OUT)
        masked = jnp.where(ind[:, :, None] != 0, y3, 0.0)
        obuf[buf] = masked.reshape(CHUNK, D_OUT)
        _out_copy(o_hbm, obuf, osem, i).start()
        if i + NBUF < NCHUNK:
            _in_copy(data_hbm, dbuf, isem, i + NBUF).start()
    for i in range(max(0, NCHUNK - NBUF), NCHUNK):
        _out_copy(o_hbm, obuf, osem, i).wait()


def kernel(indicators, data, W, b):
    ind2d = indicators.reshape(128, 128)
    b2d = b.reshape(1, D_OUT)
    hbm = pl.BlockSpec(memory_space=pltpu.MemorySpace.HBM)
    return pl.pallas_call(
        _body,
        in_specs=[hbm, hbm, hbm, hbm],
        out_specs=hbm,
        out_shape=jax.ShapeDtypeStruct((B, D_OUT), jnp.float32),
        scratch_shapes=[
            pltpu.VMEM((128, 128), jnp.int32),
            pltpu.VMEM((D_IN, D_OUT), jnp.float32),
            pltpu.VMEM((1, D_OUT), jnp.float32),
            pltpu.VMEM((NBUF, CHUNK, D_IN), jnp.float32),
            pltpu.VMEM((NBUF, CHUNK, D_OUT), jnp.float32),
            pltpu.SemaphoreType.DMA((3,)),
            pltpu.SemaphoreType.DMA((NBUF,)),
            pltpu.SemaphoreType.DMA((NBUF,)),
        ],
    )(ind2d, data, W, b2d)


# speedup vs baseline: 1.1194x; 1.0999x over previous
import jax
import jax.numpy as jnp
from jax.experimental import pallas as pl
from jax.experimental.pallas import tpu as pltpu

B = 16384
D_IN = 128
D_OUT = 128
CHUNK = 8192
NBUF = 2
NCHUNK = B // CHUNK
_Q = CHUNK // 128


def _in_copy(data_hbm, dbuf, isem, i):
    buf = i % NBUF
    return pltpu.make_async_copy(
        data_hbm.at[pl.ds(i * CHUNK, CHUNK), :], dbuf.at[buf], isem.at[buf]
    )


def _out_copy(o_hbm, obuf, osem, i):
    buf = i % NBUF
    return pltpu.make_async_copy(
        obuf.at[buf], o_hbm.at[pl.ds(i * CHUNK, CHUNK), :], osem.at[buf]
    )


def _body(ind_hbm, data_hbm, w_hbm, b_hbm, o_hbm,
          ind_v, w_v, b_v, dbuf, obuf, csem, isem, osem):
    cp_ind = pltpu.make_async_copy(ind_hbm, ind_v, csem.at[0])
    cp_w = pltpu.make_async_copy(w_hbm, w_v, csem.at[1])
    cp_b = pltpu.make_async_copy(b_hbm, b_v, csem.at[2])
    cp_ind.start()
    cp_w.start()
    cp_b.start()
    for k in range(min(NBUF, NCHUNK)):
        _in_copy(data_hbm, dbuf, isem, k).start()
    cp_ind.wait()
    cp_w.wait()
    cp_b.wait()
    for i in range(NCHUNK):
        buf = i % NBUF
        _in_copy(data_hbm, dbuf, isem, i).wait()
        if i >= NBUF:
            _out_copy(o_hbm, obuf, osem, i - NBUF).wait()
        x = dbuf[buf]
        y = jnp.dot(x, w_v[...], preferred_element_type=jnp.float32)
        y = y + b_v[...]
        ind = ind_v[pl.ds(i * _Q, _Q), :]
        y3 = y.reshape(_Q, 128, D_OUT)
        masked = jnp.where(ind[:, :, None] != 0, y3, 0.0)
        obuf[buf] = masked.reshape(CHUNK, D_OUT)
        _out_copy(o_hbm, obuf, osem, i).start()
        if i + NBUF < NCHUNK:
            _in_copy(data_hbm, dbuf, isem, i + NBUF).start()
    for i in range(max(0, NCHUNK - NBUF), NCHUNK):
        _out_copy(o_hbm, obuf, osem, i).wait()


def kernel(indicators, data, W, b):
    ind2d = indicators.reshape(128, 128)
    b2d = b.reshape(1, D_OUT)
    hbm = pl.BlockSpec(memory_space=pltpu.MemorySpace.HBM)
    return pl.pallas_call(
        _body,
        in_specs=[hbm, hbm, hbm, hbm],
        out_specs=hbm,
        out_shape=jax.ShapeDtypeStruct((B, D_OUT), jnp.float32),
        scratch_shapes=[
            pltpu.VMEM((128, 128), jnp.int32),
            pltpu.VMEM((D_IN, D_OUT), jnp.float32),
            pltpu.VMEM((1, D_OUT), jnp.float32),
            pltpu.VMEM((NBUF, CHUNK, D_IN), jnp.float32),
            pltpu.VMEM((NBUF, CHUNK, D_OUT), jnp.float32),
            pltpu.SemaphoreType.DMA((3,)),
            pltpu.SemaphoreType.DMA((NBUF,)),
            pltpu.SemaphoreType.DMA((NBUF,)),
        ],
    )(ind2d, data, W, b2d)


# speedup vs baseline: 1.1550x; 1.0318x over previous
import jax
import jax.numpy as jnp
from jax.experimental import pallas as pl
from jax.experimental.pallas import tpu as pltpu

B = 16384
D_IN = 128
D_OUT = 128
CHUNK = 4096
NBUF = 4
NCHUNK = B // CHUNK
_Q = CHUNK // 128


def _in_copy(data_hbm, dbuf, isem, i):
    buf = i % NBUF
    return pltpu.make_async_copy(
        data_hbm.at[pl.ds(i * CHUNK, CHUNK), :], dbuf.at[buf], isem.at[buf]
    )


def _out_copy(o_hbm, obuf, osem, i):
    buf = i % NBUF
    return pltpu.make_async_copy(
        obuf.at[buf], o_hbm.at[pl.ds(i * CHUNK, CHUNK), :], osem.at[buf]
    )


def _body(ind_hbm, data_hbm, w_hbm, b_hbm, o_hbm,
          ind_v, w_v, b_v, dbuf, obuf, csem, isem, osem):
    cp_ind = pltpu.make_async_copy(ind_hbm, ind_v, csem.at[0])
    cp_w = pltpu.make_async_copy(w_hbm, w_v, csem.at[1])
    cp_b = pltpu.make_async_copy(b_hbm, b_v, csem.at[2])
    cp_ind.start()
    cp_w.start()
    cp_b.start()
    for k in range(min(NBUF, NCHUNK)):
        _in_copy(data_hbm, dbuf, isem, k).start()
    cp_ind.wait()
    cp_w.wait()
    cp_b.wait()
    for i in range(NCHUNK):
        buf = i % NBUF
        _in_copy(data_hbm, dbuf, isem, i).wait()
        if i >= NBUF:
            _out_copy(o_hbm, obuf, osem, i - NBUF).wait()
        x = dbuf[buf]
        y = jnp.dot(x, w_v[...], preferred_element_type=jnp.float32)
        y = y + b_v[...]
        ind = ind_v[pl.ds(i * _Q, _Q), :]
        y3 = y.reshape(_Q, 128, D_OUT)
        masked = jnp.where(ind[:, :, None] != 0, y3, 0.0)
        obuf[buf] = masked.reshape(CHUNK, D_OUT)
        _out_copy(o_hbm, obuf, osem, i).start()
        if i + NBUF < NCHUNK:
            _in_copy(data_hbm, dbuf, isem, i + NBUF).start()
    for i in range(max(0, NCHUNK - NBUF), NCHUNK):
        _out_copy(o_hbm, obuf, osem, i).wait()


def kernel(indicators, data, W, b):
    ind2d = indicators.reshape(128, 128)
    b2d = b.reshape(1, D_OUT)
    hbm = pl.BlockSpec(memory_space=pltpu.MemorySpace.HBM)
    return pl.pallas_call(
        _body,
        in_specs=[hbm, hbm, hbm, hbm],
        out_specs=hbm,
        out_shape=jax.ShapeDtypeStruct((B, D_OUT), jnp.float32),
        scratch_shapes=[
            pltpu.VMEM((128, 128), jnp.int32),
            pltpu.VMEM((D_IN, D_OUT), jnp.float32),
            pltpu.VMEM((1, D_OUT), jnp.float32),
            pltpu.VMEM((NBUF, CHUNK, D_IN), jnp.float32),
            pltpu.VMEM((NBUF, CHUNK, D_OUT), jnp.float32),
            pltpu.SemaphoreType.DMA((3,)),
            pltpu.SemaphoreType.DMA((NBUF,)),
            pltpu.SemaphoreType.DMA((NBUF,)),
        ],
    )(ind2d, data, W, b2d)
